# baseline (speedup 1.0000x reference)
import functools

import jax
import jax.numpy as jnp
from jax import lax
from jax.experimental import pallas as pl
from jax.experimental.pallas import tpu as pltpu
from jax.experimental.pallas import tpu_sc as plsc

N = 10000
E = 320000
D = 128
NC = 2
NS = 16
NW = NC * NS
EPW = E // NW
CH = 80
NCHUNK = EPW // CH
NPAD = 10240
RPT = NPAD // NS
ZR = 128
ZD = NPAD // NS

def _worker_id():
    return lax.axis_index("s") * NC + lax.axis_index("c")


SEG = 25
NSEG = NCHUNK // SEG


def _sc_deg_body(ei5_hbm, out_hbm, acc, seg0, seg1, ones, zbuf, lsem0,
                 lsem1, ssem):
    c = lax.axis_index("c")
    s = lax.axis_index("s")
    wid = _worker_id()
    ones_offs = list(range(0, CH - 15, 16))
    if CH % 16:
        ones_offs.append(CH - 16)
    for off in ones_offs:
        ones[pl.ds(off, 16)] = jnp.ones((16,), jnp.float32)
    for k in range(ZD // 16):
        zbuf[pl.ds(16 * k, 16)] = jnp.zeros((16,), jnp.float32)
    pltpu.sync_copy(zbuf, acc.at[pl.ds(s * ZD, ZD)])
    segs = (seg0, seg1)
    lsems = (lsem0, lsem1)
    pltpu.async_copy(ei5_hbm.at[1, wid, 0], seg0, lsem0)
    plsc.subcore_barrier()

    for b in range(NSEG):
        buf, sem = segs[b % 2], lsems[b % 2]
        if b + 1 < NSEG:
            pltpu.async_copy(
                ei5_hbm.at[1, wid, b + 1], segs[(b + 1) % 2], lsems[(b + 1) % 2]
            )
        pltpu.make_async_copy(ei5_hbm.at[1, wid, b], buf, sem).wait()

        def fire(j, carry):
            pltpu.async_copy(ones, acc.at[buf.at[j]], ssem, add=True)
            return carry

        def drain(j, carry):
            pltpu.make_async_copy(ones, acc.at[buf.at[j]], ssem).wait()
            return carry

        lax.fori_loop(0, SEG, fire, 0)
        lax.fori_loop(0, SEG, drain, 0)
    plsc.subcore_barrier()
    pltpu.sync_copy(acc.at[pl.ds(s * ZD, ZD)], out_hbm.at[c, pl.ds(s * ZD, ZD)])


def _sc_pass_body(tab_hbm, ei4_hbm, dst1_hbm, out_hbm, acc, sidx, didx0,
                  didx1, rows0, rows1, gsem0, gsem1, ssem0, ssem1, lsem0,
                  lsem1):
    c = lax.axis_index("c")
    s = lax.axis_index("s")
    wid = _worker_id()

    def zfill(i, carry):
        for k in range(D // 16):
            rows0[i, pl.ds(16 * k, 16)] = jnp.zeros((16,), jnp.float32)
        return carry

    lax.fori_loop(0, CH, zfill, 0)
    for r in range(RPT // CH):
        pltpu.sync_copy(rows0, acc.at[pl.ds(s * RPT + r * CH, CH)])
    pltpu.sync_copy(ei4_hbm.at[0, wid], sidx)

    def dst_slice(j):
        return dst1_hbm.at[pl.ds(pl.multiple_of(wid * EPW + j * CH, 8), CH)]

    pltpu.sync_copy(dst_slice(0), didx0)
    pltpu.sync_copy(dst_slice(1), didx1)
    plsc.subcore_barrier()

    def gather(j, buf, sem):
        pltpu.async_copy(tab_hbm.at[sidx.at[j]], buf, sem)

    def gather_wait(j, buf, sem):
        pltpu.make_async_copy(tab_hbm.at[sidx.at[j]], buf, sem).wait()

    def scat(buf, dbuf, sem):
        pltpu.async_copy(buf, acc.at[dbuf], sem, add=True)

    def scat_wait(buf, dbuf, sem):
        pltpu.make_async_copy(buf, acc.at[dbuf], sem).wait()

    def dload(j, dbuf, sem):
        pltpu.async_copy(dst_slice(j), dbuf, sem)

    def dload_wait(j, dbuf, sem):
        pltpu.make_async_copy(dst_slice(j), dbuf, sem).wait()

    gather(0, rows0, gsem0)

    def pair(i, carry):
        j0 = 2 * i

        @pl.when(i >= 1)
        def _():
            scat_wait(rows1, didx1, ssem1)
            dload(j0 + 1, didx1, lsem1)

        gather(j0 + 1, rows1, gsem1)
        gather_wait(j0, rows0, gsem0)

        @pl.when(i >= 1)
        def _():
            dload_wait(j0, didx0, lsem0)

        scat(rows0, didx0, ssem0)
        gather_wait(j0 + 1, rows1, gsem1)

        @pl.when(i >= 1)
        def _():
            dload_wait(j0 + 1, didx1, lsem1)

        scat(rows1, didx1, ssem1)
        scat_wait(rows0, didx0, ssem0)
        dload(j0 + 2, didx0, lsem0)
        gather(j0 + 2, rows0, gsem0)
        return carry

    last = NCHUNK - 1
    lax.fori_loop(0, (NCHUNK - 1) // 2, pair, 0)
    scat_wait(rows1, didx1, ssem1)
    gather_wait(last, rows0, gsem0)
    dload_wait(last, didx0, lsem0)
    scat(rows0, didx0, ssem0)
    scat_wait(rows0, didx0, ssem0)
    plsc.subcore_barrier()
    pltpu.sync_copy(
        acc.at[pl.ds(s * RPT, RPT)], out_hbm.at[c, pl.ds(s * RPT, RPT)]
    )


@functools.lru_cache(maxsize=None)
def _get_sc_kernels():
    mesh = plsc.VectorSubcoreMesh(
        core_axis_name="c", subcore_axis_name="s",
        num_cores=NC, num_subcores=NS,
    )
    sc_deg = pl.kernel(
        _sc_deg_body,
        out_type=jax.ShapeDtypeStruct((NC, NPAD), jnp.float32),
        mesh=mesh,
        scratch_types=[
            pltpu.VMEM_SHARED((NPAD,), jnp.float32),
            pltpu.VMEM((SEG, CH), jnp.int32),
            pltpu.VMEM((SEG, CH), jnp.int32),
            pltpu.VMEM((CH,), jnp.float32),
            pltpu.VMEM((ZD,), jnp.float32),
            pltpu.SemaphoreType.DMA,
            pltpu.SemaphoreType.DMA,
            pltpu.SemaphoreType.DMA,
        ],
    )
    sc_pass = pl.kernel(
        _sc_pass_body,
        out_type=jax.ShapeDtypeStruct((NC, NPAD, D), jnp.float32),
        mesh=mesh,
        scratch_types=[
            pltpu.VMEM_SHARED((NPAD, D), jnp.float32),
            pltpu.VMEM((NCHUNK, CH), jnp.int32),
            pltpu.VMEM((CH,), jnp.int32),
            pltpu.VMEM((CH,), jnp.int32),
            pltpu.VMEM((CH, D), jnp.float32),
            pltpu.VMEM((CH, D), jnp.float32),
            pltpu.SemaphoreType.DMA,
            pltpu.SemaphoreType.DMA,
            pltpu.SemaphoreType.DMA,
            pltpu.SemaphoreType.DMA,
            pltpu.SemaphoreType.DMA,
            pltpu.SemaphoreType.DMA,
        ],
    )
    return sc_deg, sc_pass


RB = 1024
GRID = NPAD // RB
_CT = (((1,), (1,)), ((), ()))
_F32 = jnp.float32


def _tc_l1_body(x_ref, wli_ref, bli_ref, wlos_ref, blos_ref, l1_ref):
    x = x_ref[...]
    t = lax.dot_general(x, wli_ref[...], _CT, preferred_element_type=_F32)
    t = jnp.maximum(t + bli_ref[...][None, :], 0.0)
    wlos = wlos_ref[...]
    l1_ref[...] = (
        lax.dot_general(x, wlos[:, :D], _CT, preferred_element_type=_F32)
        + lax.dot_general(t, wlos[:, D:], _CT, preferred_element_type=_F32)
        + blos_ref[...][None, :]
    )


_tc_l1 = pl.pallas_call(
    _tc_l1_body,
    grid=(GRID,),
    in_specs=[
        pl.BlockSpec((RB, D), lambda b: (b, 0)),
        pl.BlockSpec((2 * D, D), lambda b: (0, 0)),
        pl.BlockSpec((2 * D,), lambda b: (0,)),
        pl.BlockSpec((D, 3 * D), lambda b: (0, 0)),
        pl.BlockSpec((D,), lambda b: (0,)),
    ],
    out_specs=pl.BlockSpec((RB, D), lambda b: (b, 0)),
    out_shape=jax.ShapeDtypeStruct((N, D), _F32),
)


def _tc_a_body(x_ref, dp_ref, wg1_ref, wg2_ref, zt_ref, dinv_ref):
    x = x_ref[...]
    deg = dp_ref[0] + dp_ref[1] + 1.0
    dinv = lax.rsqrt(deg)
    dinv_ref[...] = dinv
    wc = lax.dot_general(
        wg2_ref[...], wg1_ref[...], (((1,), (0,)), ((), ())),
        preferred_element_type=_F32,
    )
    zt_ref[...] = dinv[:, None] * lax.dot_general(
        x, wc, _CT, preferred_element_type=_F32
    )


_tc_a = pl.pallas_call(
    _tc_a_body,
    grid=(GRID,),
    in_specs=[
        pl.BlockSpec((RB, D), lambda b: (b, 0)),
        pl.BlockSpec((NC, RB), lambda b: (0, b)),
        pl.BlockSpec((2 * D, D), lambda b: (0, 0)),
        pl.BlockSpec((D, 2 * D), lambda b: (0, 0)),
    ],
    out_specs=[
        pl.BlockSpec((RB, D), lambda b: (b, 0)),
        pl.BlockSpec((RB,), lambda b: (b,)),
    ],
    out_shape=[
        jax.ShapeDtypeStruct((N, D), _F32),
        jax.ShapeDtypeStruct((NPAD,), _F32),
    ],
)


def _tc_b_body(acc_ref, zt_ref, dinv_ref, wg2_ref, bg1_ref, zt2_ref):
    dinv = dinv_ref[...][:, None]
    cvec = lax.dot_general(
        bg1_ref[...][None, :], wg2_ref[...], _CT, preferred_element_type=_F32
    )
    y1p = dinv * (acc_ref[0] + acc_ref[1] + zt_ref[...]) + cvec
    zt2_ref[...] = dinv * y1p


_tc_b = pl.pallas_call(
    _tc_b_body,
    grid=(GRID,),
    in_specs=[
        pl.BlockSpec((NC, RB, D), lambda b: (0, b, 0)),
        pl.BlockSpec((RB, D), lambda b: (b, 0)),
        pl.BlockSpec((RB,), lambda b: (b,)),
        pl.BlockSpec((D, 2 * D), lambda b: (0, 0)),
        pl.BlockSpec((2 * D,), lambda b: (0,)),
    ],
    out_specs=pl.BlockSpec((RB, D), lambda b: (b, 0)),
    out_shape=jax.ShapeDtypeStruct((N, D), _F32),
)


def _tc_c_body(acc_ref, zt2_ref, dinv_ref, xn_ref, wlo_ref, blo_ref,
               bg2_ref, x2_ref):
    dinv = dinv_ref[...][:, None]
    g2 = dinv * (acc_ref[0] + acc_ref[1] + zt2_ref[...]) + bg2_ref[...][None, :]
    wlo = wlo_ref[...]
    x2_ref[...] = (
        lax.dot_general(xn_ref[...], wlo[:, :D], _CT, preferred_element_type=_F32)
        + lax.dot_general(g2, wlo[:, D:], _CT, preferred_element_type=_F32)
        + blo_ref[...][None, :]
    )


_tc_c = pl.pallas_call(
    _tc_c_body,
    grid=(GRID,),
    in_specs=[
        pl.BlockSpec((NC, RB, D), lambda b: (0, b, 0)),
        pl.BlockSpec((RB, D), lambda b: (b, 0)),
        pl.BlockSpec((RB,), lambda b: (b,)),
        pl.BlockSpec((RB, D), lambda b: (b, 0)),
        pl.BlockSpec((D, 2 * D), lambda b: (0, 0)),
        pl.BlockSpec((D,), lambda b: (0,)),
        pl.BlockSpec((D,), lambda b: (0,)),
    ],
    out_specs=pl.BlockSpec((RB, D), lambda b: (b, 0)),
    out_shape=jax.ShapeDtypeStruct((N, D), _F32),
)


def kernel(x_self, x_neighbor, edge_index, W_lin_in, b_lin_in, W_los, b_los,
           W_g1, b_g1, W_g2, b_g2, W_lo, b_lo):
    _sc_deg, _sc_pass = _get_sc_kernels()
    ei4 = edge_index.reshape(2, NW, NCHUNK, CH)
    ei5 = edge_index.reshape(2, NW, NSEG, SEG, CH)
    dst1 = edge_index[1]
    deg_part = _sc_deg(ei5)
    l1 = _tc_l1(x_self, W_lin_in, b_lin_in, W_los, b_los)
    zt, dinv = _tc_a(x_self, deg_part, W_g1, W_g2)
    acc1 = _sc_pass(zt, ei4, dst1)
    zt2 = _tc_b(acc1, zt, dinv, W_g2, b_g1)
    acc2 = _sc_pass(zt2, ei4, dst1)
    x2 = _tc_c(acc2, zt2, dinv, x_neighbor, W_lo, b_lo, b_g2)
    return (l1, x2)


# speedup vs baseline: 1.2258x; 1.2258x over previous
import functools

import jax
import jax.numpy as jnp
from jax import lax
from jax.experimental import pallas as pl
from jax.experimental.pallas import tpu as pltpu
from jax.experimental.pallas import tpu_sc as plsc

N = 10000
E = 320000
D = 128
NC = 2
NS = 16
NW = NC * NS
EPW = E // NW
CH = 80
NCHUNK = EPW // CH
NPAD = 10240
RPT = NPAD // NS
ZR = 128
ZD = NPAD // NS

def _worker_id():
    return lax.axis_index("s") * NC + lax.axis_index("c")


SEG = 25
NSEG = NCHUNK // SEG


def _sc_deg_body(ei5_hbm, out_hbm, acc, seg0, seg1, ones, zbuf, lsem0,
                 lsem1, ssem):
    c = lax.axis_index("c")
    s = lax.axis_index("s")
    wid = _worker_id()
    ones_offs = list(range(0, CH - 15, 16))
    if CH % 16:
        ones_offs.append(CH - 16)
    for off in ones_offs:
        ones[pl.ds(off, 16)] = jnp.ones((16,), jnp.float32)
    for k in range(ZD // 16):
        zbuf[pl.ds(16 * k, 16)] = jnp.zeros((16,), jnp.float32)
    pltpu.sync_copy(zbuf, acc.at[pl.ds(s * ZD, ZD)])
    segs = (seg0, seg1)
    lsems = (lsem0, lsem1)
    pltpu.async_copy(ei5_hbm.at[1, wid, 0], seg0, lsem0)
    plsc.subcore_barrier()

    for b in range(NSEG):
        buf, sem = segs[b % 2], lsems[b % 2]
        if b + 1 < NSEG:
            pltpu.async_copy(
                ei5_hbm.at[1, wid, b + 1], segs[(b + 1) % 2], lsems[(b + 1) % 2]
            )
        pltpu.make_async_copy(ei5_hbm.at[1, wid, b], buf, sem).wait()

        def fire(j, carry):
            pltpu.async_copy(ones, acc.at[buf.at[j]], ssem, add=True)
            return carry

        def drain(j, carry):
            pltpu.make_async_copy(ones, acc.at[buf.at[j]], ssem).wait()
            return carry

        lax.fori_loop(0, SEG, fire, 0)
        lax.fori_loop(0, SEG, drain, 0)
    plsc.subcore_barrier()
    pltpu.sync_copy(acc.at[pl.ds(s * ZD, ZD)], out_hbm.at[c, pl.ds(s * ZD, ZD)])


def _sc_pass_body(tab_hbm, ei4_hbm, dst1_hbm, out_hbm, acc, sidx, didx0,
                  didx1, rows0, rows1, gsem0, gsem1, ssem0, ssem1, lsem0,
                  lsem1):
    c = lax.axis_index("c")
    s = lax.axis_index("s")
    wid = _worker_id()

    def zfill(i, carry):
        for k in range(D // 16):
            rows0[i, pl.ds(16 * k, 16)] = jnp.zeros((16,), jnp.float32)
        return carry

    lax.fori_loop(0, CH, zfill, 0)
    for r in range(RPT // CH):
        pltpu.sync_copy(rows0, acc.at[pl.ds(s * RPT + r * CH, CH)])
    pltpu.sync_copy(ei4_hbm.at[0, wid], sidx)

    def dst_slice(j):
        return dst1_hbm.at[pl.ds(pl.multiple_of(wid * EPW + j * CH, 8), CH)]

    pltpu.sync_copy(dst_slice(0), didx0)
    pltpu.sync_copy(dst_slice(1), didx1)
    plsc.subcore_barrier()

    def gather(j, buf, sem):
        pltpu.async_copy(tab_hbm.at[sidx.at[j]], buf, sem)

    def gather_wait(j, buf, sem):
        pltpu.make_async_copy(tab_hbm.at[sidx.at[j]], buf, sem).wait()

    def scat(buf, dbuf, sem):
        pltpu.async_copy(buf, acc.at[dbuf], sem, add=True)

    def scat_wait(buf, dbuf, sem):
        pltpu.make_async_copy(buf, acc.at[dbuf], sem).wait()

    def dload(j, dbuf, sem):
        pltpu.async_copy(dst_slice(j), dbuf, sem)

    def dload_wait(j, dbuf, sem):
        pltpu.make_async_copy(dst_slice(j), dbuf, sem).wait()

    gather(0, rows0, gsem0)

    def pair(i, carry):
        j0 = 2 * i

        @pl.when(i >= 1)
        def _():
            scat_wait(rows1, didx1, ssem1)
            dload(j0 + 1, didx1, lsem1)

        gather(j0 + 1, rows1, gsem1)
        gather_wait(j0, rows0, gsem0)

        @pl.when(i >= 1)
        def _():
            dload_wait(j0, didx0, lsem0)

        scat(rows0, didx0, ssem0)
        scat_wait(rows0, didx0, ssem0)
        dload(j0 + 2, didx0, lsem0)
        gather(j0 + 2, rows0, gsem0)

        @pl.when(i >= 1)
        def _():
            dload_wait(j0 + 1, didx1, lsem1)

        gather_wait(j0 + 1, rows1, gsem1)
        scat(rows1, didx1, ssem1)
        return carry

    last = NCHUNK - 1
    lax.fori_loop(0, (NCHUNK - 1) // 2, pair, 0)
    scat_wait(rows1, didx1, ssem1)
    gather_wait(last, rows0, gsem0)
    dload_wait(last, didx0, lsem0)
    scat(rows0, didx0, ssem0)
    scat_wait(rows0, didx0, ssem0)
    plsc.subcore_barrier()
    pltpu.sync_copy(
        acc.at[pl.ds(s * RPT, RPT)], out_hbm.at[c, pl.ds(s * RPT, RPT)]
    )


@functools.lru_cache(maxsize=None)
def _get_sc_kernels():
    mesh = plsc.VectorSubcoreMesh(
        core_axis_name="c", subcore_axis_name="s",
        num_cores=NC, num_subcores=NS,
    )
    sc_deg = pl.kernel(
        _sc_deg_body,
        out_type=jax.ShapeDtypeStruct((NC, NPAD), jnp.float32),
        mesh=mesh,
        scratch_types=[
            pltpu.VMEM_SHARED((NPAD,), jnp.float32),
            pltpu.VMEM((SEG, CH), jnp.int32),
            pltpu.VMEM((SEG, CH), jnp.int32),
            pltpu.VMEM((CH,), jnp.float32),
            pltpu.VMEM((ZD,), jnp.float32),
            pltpu.SemaphoreType.DMA,
            pltpu.SemaphoreType.DMA,
            pltpu.SemaphoreType.DMA,
        ],
    )
    sc_pass = pl.kernel(
        _sc_pass_body,
        out_type=jax.ShapeDtypeStruct((NC, NPAD, D), jnp.float32),
        mesh=mesh,
        scratch_types=[
            pltpu.VMEM_SHARED((NPAD, D), jnp.float32),
            pltpu.VMEM((NCHUNK, CH), jnp.int32),
            pltpu.VMEM((CH,), jnp.int32),
            pltpu.VMEM((CH,), jnp.int32),
            pltpu.VMEM((CH, D), jnp.float32),
            pltpu.VMEM((CH, D), jnp.float32),
            pltpu.SemaphoreType.DMA,
            pltpu.SemaphoreType.DMA,
            pltpu.SemaphoreType.DMA,
            pltpu.SemaphoreType.DMA,
            pltpu.SemaphoreType.DMA,
            pltpu.SemaphoreType.DMA,
        ],
    )
    return sc_deg, sc_pass


RB = 1024
GRID = NPAD // RB
_CT = (((1,), (1,)), ((), ()))
_F32 = jnp.float32


def _tc_l1_body(x_ref, wli_ref, bli_ref, wlos_ref, blos_ref, l1_ref):
    x = x_ref[...]
    t = lax.dot_general(x, wli_ref[...], _CT, preferred_element_type=_F32)
    t = jnp.maximum(t + bli_ref[...][None, :], 0.0)
    wlos = wlos_ref[...]
    l1_ref[...] = (
        lax.dot_general(x, wlos[:, :D], _CT, preferred_element_type=_F32)
        + lax.dot_general(t, wlos[:, D:], _CT, preferred_element_type=_F32)
        + blos_ref[...][None, :]
    )


_tc_l1 = pl.pallas_call(
    _tc_l1_body,
    grid=(GRID,),
    in_specs=[
        pl.BlockSpec((RB, D), lambda b: (b, 0)),
        pl.BlockSpec((2 * D, D), lambda b: (0, 0)),
        pl.BlockSpec((2 * D,), lambda b: (0,)),
        pl.BlockSpec((D, 3 * D), lambda b: (0, 0)),
        pl.BlockSpec((D,), lambda b: (0,)),
    ],
    out_specs=pl.BlockSpec((RB, D), lambda b: (b, 0)),
    out_shape=jax.ShapeDtypeStruct((N, D), _F32),
)


def _tc_a_body(x_ref, dp_ref, wg1_ref, wg2_ref, zt_ref, dinv_ref):
    x = x_ref[...]
    deg = dp_ref[0] + dp_ref[1] + 1.0
    dinv = lax.rsqrt(deg)
    dinv_ref[...] = dinv
    wc = lax.dot_general(
        wg2_ref[...], wg1_ref[...], (((1,), (0,)), ((), ())),
        preferred_element_type=_F32,
    )
    zt_ref[...] = dinv[:, None] * lax.dot_general(
        x, wc, _CT, preferred_element_type=_F32
    )


_tc_a = pl.pallas_call(
    _tc_a_body,
    grid=(GRID,),
    in_specs=[
        pl.BlockSpec((RB, D), lambda b: (b, 0)),
        pl.BlockSpec((NC, RB), lambda b: (0, b)),
        pl.BlockSpec((2 * D, D), lambda b: (0, 0)),
        pl.BlockSpec((D, 2 * D), lambda b: (0, 0)),
    ],
    out_specs=[
        pl.BlockSpec((RB, D), lambda b: (b, 0)),
        pl.BlockSpec((RB,), lambda b: (b,)),
    ],
    out_shape=[
        jax.ShapeDtypeStruct((N, D), _F32),
        jax.ShapeDtypeStruct((NPAD,), _F32),
    ],
)


def _tc_b_body(acc_ref, zt_ref, dinv_ref, wg2_ref, bg1_ref, zt2_ref):
    dinv = dinv_ref[...][:, None]
    cvec = lax.dot_general(
        bg1_ref[...][None, :], wg2_ref[...], _CT, preferred_element_type=_F32
    )
    y1p = dinv * (acc_ref[0] + acc_ref[1] + zt_ref[...]) + cvec
    zt2_ref[...] = dinv * y1p


_tc_b = pl.pallas_call(
    _tc_b_body,
    grid=(GRID,),
    in_specs=[
        pl.BlockSpec((NC, RB, D), lambda b: (0, b, 0)),
        pl.BlockSpec((RB, D), lambda b: (b, 0)),
        pl.BlockSpec((RB,), lambda b: (b,)),
        pl.BlockSpec((D, 2 * D), lambda b: (0, 0)),
        pl.BlockSpec((2 * D,), lambda b: (0,)),
    ],
    out_specs=pl.BlockSpec((RB, D), lambda b: (b, 0)),
    out_shape=jax.ShapeDtypeStruct((N, D), _F32),
)


def _tc_c_body(acc_ref, zt2_ref, dinv_ref, xn_ref, wlo_ref, blo_ref,
               bg2_ref, x2_ref):
    dinv = dinv_ref[...][:, None]
    g2 = dinv * (acc_ref[0] + acc_ref[1] + zt2_ref[...]) + bg2_ref[...][None, :]
    wlo = wlo_ref[...]
    x2_ref[...] = (
        lax.dot_general(xn_ref[...], wlo[:, :D], _CT, preferred_element_type=_F32)
        + lax.dot_general(g2, wlo[:, D:], _CT, preferred_element_type=_F32)
        + blo_ref[...][None, :]
    )


_tc_c = pl.pallas_call(
    _tc_c_body,
    grid=(GRID,),
    in_specs=[
        pl.BlockSpec((NC, RB, D), lambda b: (0, b, 0)),
        pl.BlockSpec((RB, D), lambda b: (b, 0)),
        pl.BlockSpec((RB,), lambda b: (b,)),
        pl.BlockSpec((RB, D), lambda b: (b, 0)),
        pl.BlockSpec((D, 2 * D), lambda b: (0, 0)),
        pl.BlockSpec((D,), lambda b: (0,)),
        pl.BlockSpec((D,), lambda b: (0,)),
    ],
    out_specs=pl.BlockSpec((RB, D), lambda b: (b, 0)),
    out_shape=jax.ShapeDtypeStruct((N, D), _F32),
)


def kernel(x_self, x_neighbor, edge_index, W_lin_in, b_lin_in, W_los, b_los,
           W_g1, b_g1, W_g2, b_g2, W_lo, b_lo):
    _sc_deg, _sc_pass = _get_sc_kernels()
    ei4 = edge_index.reshape(2, NW, NCHUNK, CH)
    ei5 = edge_index.reshape(2, NW, NSEG, SEG, CH)
    dst1 = edge_index[1]
    deg_part = _sc_deg(ei5)
    l1 = _tc_l1(x_self, W_lin_in, b_lin_in, W_los, b_los)
    zt, dinv = _tc_a(x_self, deg_part, W_g1, W_g2)
    acc1 = _sc_pass(zt, ei4, dst1)
    zt2 = _tc_b(acc1, zt, dinv, W_g2, b_g1)
    acc2 = _sc_pass(zt2, ei4, dst1)
    x2 = _tc_c(acc2, zt2, dinv, x_neighbor, W_lo, b_lo, b_g2)
    return (l1, x2)


# speedup vs baseline: 1.2462x; 1.0167x over previous
import functools

import jax
import jax.numpy as jnp
from jax import lax
from jax.experimental import pallas as pl
from jax.experimental.pallas import tpu as pltpu
from jax.experimental.pallas import tpu_sc as plsc

N = 10000
E = 320000
D = 128
NC = 2
NS = 16
NW = NC * NS
EPW = E // NW
CH = 80
NCHUNK = EPW // CH
NPAD = 10240
RPT = NPAD // NS
ZR = 128
ZD = NPAD // NS

def _worker_id():
    return lax.axis_index("s") * NC + lax.axis_index("c")


SEG = 25
NSEG = NCHUNK // SEG


def _sc_deg_body(ei5_hbm, out_hbm, acc, seg0, seg1, ones, zbuf, lsem0,
                 lsem1, ssem):
    c = lax.axis_index("c")
    s = lax.axis_index("s")
    wid = _worker_id()
    ones_offs = list(range(0, CH - 15, 16))
    if CH % 16:
        ones_offs.append(CH - 16)
    for off in ones_offs:
        ones[pl.ds(off, 16)] = jnp.ones((16,), jnp.float32)
    for k in range(ZD // 16):
        zbuf[pl.ds(16 * k, 16)] = jnp.zeros((16,), jnp.float32)
    pltpu.sync_copy(zbuf, acc.at[pl.ds(s * ZD, ZD)])
    segs = (seg0, seg1)
    lsems = (lsem0, lsem1)
    pltpu.async_copy(ei5_hbm.at[1, wid, 0], seg0, lsem0)
    plsc.subcore_barrier()

    for b in range(NSEG):
        buf, sem = segs[b % 2], lsems[b % 2]
        if b + 1 < NSEG:
            pltpu.async_copy(
                ei5_hbm.at[1, wid, b + 1], segs[(b + 1) % 2], lsems[(b + 1) % 2]
            )
        pltpu.make_async_copy(ei5_hbm.at[1, wid, b], buf, sem).wait()

        def fire(j, carry):
            pltpu.async_copy(ones, acc.at[buf.at[j]], ssem, add=True)
            return carry

        def drain(j, carry):
            pltpu.make_async_copy(ones, acc.at[buf.at[j]], ssem).wait()
            return carry

        lax.fori_loop(0, SEG, fire, 0)
        lax.fori_loop(0, SEG, drain, 0)
    plsc.subcore_barrier()
    pltpu.sync_copy(acc.at[pl.ds(s * ZD, ZD)], out_hbm.at[c, pl.ds(s * ZD, ZD)])


def _sc_pass_body(tab_hbm, ei4_hbm, dst1_hbm, out_hbm, acc, sidx, didx0,
                  didx1, rows0, rows1, gsem0, gsem1, ssem0, ssem1, lsem0,
                  lsem1):
    c = lax.axis_index("c")
    s = lax.axis_index("s")
    wid = _worker_id()

    def zfill(i, carry):
        for k in range(D // 16):
            rows0[i, pl.ds(16 * k, 16)] = jnp.zeros((16,), jnp.float32)
        return carry

    def dst_slice(j):
        return dst1_hbm.at[pl.ds(pl.multiple_of(wid * EPW + j * CH, 8), CH)]

    lax.fori_loop(0, CH, zfill, 0)
    for r in range(RPT // CH):
        pltpu.async_copy(rows0, acc.at[pl.ds(s * RPT + r * CH, CH)], ssem0)
    pltpu.async_copy(ei4_hbm.at[0, wid], sidx, gsem1)
    pltpu.async_copy(dst_slice(0), didx0, lsem0)
    pltpu.async_copy(dst_slice(1), didx1, lsem1)
    for r in range(RPT // CH):
        pltpu.make_async_copy(
            rows0, acc.at[pl.ds(s * RPT + r * CH, CH)], ssem0
        ).wait()
    pltpu.make_async_copy(ei4_hbm.at[0, wid], sidx, gsem1).wait()
    pltpu.make_async_copy(dst_slice(0), didx0, lsem0).wait()
    pltpu.make_async_copy(dst_slice(1), didx1, lsem1).wait()
    pltpu.async_copy(tab_hbm.at[sidx.at[0]], rows0, gsem0)
    plsc.subcore_barrier()

    def gather(j, buf, sem):
        pltpu.async_copy(tab_hbm.at[sidx.at[j]], buf, sem)

    def gather_wait(j, buf, sem):
        pltpu.make_async_copy(tab_hbm.at[sidx.at[j]], buf, sem).wait()

    def scat(buf, dbuf, sem):
        pltpu.async_copy(buf, acc.at[dbuf], sem, add=True)

    def scat_wait(buf, dbuf, sem):
        pltpu.make_async_copy(buf, acc.at[dbuf], sem).wait()

    def dload(j, dbuf, sem):
        pltpu.async_copy(dst_slice(j), dbuf, sem)

    def dload_wait(j, dbuf, sem):
        pltpu.make_async_copy(dst_slice(j), dbuf, sem).wait()


    def pair(i, carry):
        j0 = 2 * i

        @pl.when(i >= 1)
        def _():
            scat_wait(rows1, didx1, ssem1)
            dload(j0 + 1, didx1, lsem1)

        gather(j0 + 1, rows1, gsem1)
        gather_wait(j0, rows0, gsem0)

        @pl.when(i >= 1)
        def _():
            dload_wait(j0, didx0, lsem0)

        scat(rows0, didx0, ssem0)
        scat_wait(rows0, didx0, ssem0)
        dload(j0 + 2, didx0, lsem0)
        gather(j0 + 2, rows0, gsem0)

        @pl.when(i >= 1)
        def _():
            dload_wait(j0 + 1, didx1, lsem1)

        gather_wait(j0 + 1, rows1, gsem1)
        scat(rows1, didx1, ssem1)
        return carry

    last = NCHUNK - 1
    lax.fori_loop(0, (NCHUNK - 1) // 2, pair, 0)
    scat_wait(rows1, didx1, ssem1)
    gather_wait(last, rows0, gsem0)
    dload_wait(last, didx0, lsem0)
    scat(rows0, didx0, ssem0)
    scat_wait(rows0, didx0, ssem0)
    plsc.subcore_barrier()
    pltpu.sync_copy(
        acc.at[pl.ds(s * RPT, RPT)], out_hbm.at[c, pl.ds(s * RPT, RPT)]
    )


@functools.lru_cache(maxsize=None)
def _get_sc_kernels():
    mesh = plsc.VectorSubcoreMesh(
        core_axis_name="c", subcore_axis_name="s",
        num_cores=NC, num_subcores=NS,
    )
    sc_deg = pl.kernel(
        _sc_deg_body,
        out_type=jax.ShapeDtypeStruct((NC, NPAD), jnp.float32),
        mesh=mesh,
        scratch_types=[
            pltpu.VMEM_SHARED((NPAD,), jnp.float32),
            pltpu.VMEM((SEG, CH), jnp.int32),
            pltpu.VMEM((SEG, CH), jnp.int32),
            pltpu.VMEM((CH,), jnp.float32),
            pltpu.VMEM((ZD,), jnp.float32),
            pltpu.SemaphoreType.DMA,
            pltpu.SemaphoreType.DMA,
            pltpu.SemaphoreType.DMA,
        ],
    )
    sc_pass = pl.kernel(
        _sc_pass_body,
        out_type=jax.ShapeDtypeStruct((NC, NPAD, D), jnp.float32),
        mesh=mesh,
        scratch_types=[
            pltpu.VMEM_SHARED((NPAD, D), jnp.float32),
            pltpu.VMEM((NCHUNK, CH), jnp.int32),
            pltpu.VMEM((CH,), jnp.int32),
            pltpu.VMEM((CH,), jnp.int32),
            pltpu.VMEM((CH, D), jnp.float32),
            pltpu.VMEM((CH, D), jnp.float32),
            pltpu.SemaphoreType.DMA,
            pltpu.SemaphoreType.DMA,
            pltpu.SemaphoreType.DMA,
            pltpu.SemaphoreType.DMA,
            pltpu.SemaphoreType.DMA,
            pltpu.SemaphoreType.DMA,
        ],
    )
    return sc_deg, sc_pass


RB = 1024
GRID = NPAD // RB
_CT = (((1,), (1,)), ((), ()))
_F32 = jnp.float32


def _tc_l1_body(x_ref, wli_ref, bli_ref, wlos_ref, blos_ref, l1_ref):
    x = x_ref[...]
    t = lax.dot_general(x, wli_ref[...], _CT, preferred_element_type=_F32)
    t = jnp.maximum(t + bli_ref[...][None, :], 0.0)
    wlos = wlos_ref[...]
    l1_ref[...] = (
        lax.dot_general(x, wlos[:, :D], _CT, preferred_element_type=_F32)
        + lax.dot_general(t, wlos[:, D:], _CT, preferred_element_type=_F32)
        + blos_ref[...][None, :]
    )


_tc_l1 = pl.pallas_call(
    _tc_l1_body,
    grid=(GRID,),
    in_specs=[
        pl.BlockSpec((RB, D), lambda b: (b, 0)),
        pl.BlockSpec((2 * D, D), lambda b: (0, 0)),
        pl.BlockSpec((2 * D,), lambda b: (0,)),
        pl.BlockSpec((D, 3 * D), lambda b: (0, 0)),
        pl.BlockSpec((D,), lambda b: (0,)),
    ],
    out_specs=pl.BlockSpec((RB, D), lambda b: (b, 0)),
    out_shape=jax.ShapeDtypeStruct((N, D), _F32),
)


def _tc_a_body(x_ref, dp_ref, wg1_ref, wg2_ref, zt_ref, dinv_ref):
    x = x_ref[...]
    deg = dp_ref[0] + dp_ref[1] + 1.0
    dinv = lax.rsqrt(deg)
    dinv_ref[...] = dinv
    wc = lax.dot_general(
        wg2_ref[...], wg1_ref[...], (((1,), (0,)), ((), ())),
        preferred_element_type=_F32,
    )
    zt_ref[...] = dinv[:, None] * lax.dot_general(
        x, wc, _CT, preferred_element_type=_F32
    )


_tc_a = pl.pallas_call(
    _tc_a_body,
    grid=(GRID,),
    in_specs=[
        pl.BlockSpec((RB, D), lambda b: (b, 0)),
        pl.BlockSpec((NC, RB), lambda b: (0, b)),
        pl.BlockSpec((2 * D, D), lambda b: (0, 0)),
        pl.BlockSpec((D, 2 * D), lambda b: (0, 0)),
    ],
    out_specs=[
        pl.BlockSpec((RB, D), lambda b: (b, 0)),
        pl.BlockSpec((RB,), lambda b: (b,)),
    ],
    out_shape=[
        jax.ShapeDtypeStruct((N, D), _F32),
        jax.ShapeDtypeStruct((NPAD,), _F32),
    ],
)


def _tc_b_body(acc_ref, zt_ref, dinv_ref, wg2_ref, bg1_ref, zt2_ref):
    dinv = dinv_ref[...][:, None]
    cvec = lax.dot_general(
        bg1_ref[...][None, :], wg2_ref[...], _CT, preferred_element_type=_F32
    )
    y1p = dinv * (acc_ref[0] + acc_ref[1] + zt_ref[...]) + cvec
    zt2_ref[...] = dinv * y1p


_tc_b = pl.pallas_call(
    _tc_b_body,
    grid=(GRID,),
    in_specs=[
        pl.BlockSpec((NC, RB, D), lambda b: (0, b, 0)),
        pl.BlockSpec((RB, D), lambda b: (b, 0)),
        pl.BlockSpec((RB,), lambda b: (b,)),
        pl.BlockSpec((D, 2 * D), lambda b: (0, 0)),
        pl.BlockSpec((2 * D,), lambda b: (0,)),
    ],
    out_specs=pl.BlockSpec((RB, D), lambda b: (b, 0)),
    out_shape=jax.ShapeDtypeStruct((N, D), _F32),
)


def _tc_c_body(acc_ref, zt2_ref, dinv_ref, xn_ref, wlo_ref, blo_ref,
               bg2_ref, x2_ref):
    dinv = dinv_ref[...][:, None]
    g2 = dinv * (acc_ref[0] + acc_ref[1] + zt2_ref[...]) + bg2_ref[...][None, :]
    wlo = wlo_ref[...]
    x2_ref[...] = (
        lax.dot_general(xn_ref[...], wlo[:, :D], _CT, preferred_element_type=_F32)
        + lax.dot_general(g2, wlo[:, D:], _CT, preferred_element_type=_F32)
        + blo_ref[...][None, :]
    )


_tc_c = pl.pallas_call(
    _tc_c_body,
    grid=(GRID,),
    in_specs=[
        pl.BlockSpec((NC, RB, D), lambda b: (0, b, 0)),
        pl.BlockSpec((RB, D), lambda b: (b, 0)),
        pl.BlockSpec((RB,), lambda b: (b,)),
        pl.BlockSpec((RB, D), lambda b: (b, 0)),
        pl.BlockSpec((D, 2 * D), lambda b: (0, 0)),
        pl.BlockSpec((D,), lambda b: (0,)),
        pl.BlockSpec((D,), lambda b: (0,)),
    ],
    out_specs=pl.BlockSpec((RB, D), lambda b: (b, 0)),
    out_shape=jax.ShapeDtypeStruct((N, D), _F32),
)


def kernel(x_self, x_neighbor, edge_index, W_lin_in, b_lin_in, W_los, b_los,
           W_g1, b_g1, W_g2, b_g2, W_lo, b_lo):
    _sc_deg, _sc_pass = _get_sc_kernels()
    ei4 = edge_index.reshape(2, NW, NCHUNK, CH)
    ei5 = edge_index.reshape(2, NW, NSEG, SEG, CH)
    dst1 = edge_index[1]
    deg_part = _sc_deg(ei5)
    l1 = _tc_l1(x_self, W_lin_in, b_lin_in, W_los, b_los)
    zt, dinv = _tc_a(x_self, deg_part, W_g1, W_g2)
    acc1 = _sc_pass(zt, ei4, dst1)
    zt2 = _tc_b(acc1, zt, dinv, W_g2, b_g1)
    acc2 = _sc_pass(zt2, ei4, dst1)
    x2 = _tc_c(acc2, zt2, dinv, x_neighbor, W_lo, b_lo, b_g2)
    return (l1, x2)


# speedup vs baseline: 1.3173x; 1.0570x over previous
import functools

import jax
import jax.numpy as jnp
from jax import lax
from jax.experimental import pallas as pl
from jax.experimental.pallas import tpu as pltpu
from jax.experimental.pallas import tpu_sc as plsc

N = 10000
E = 320000
D = 128
NC = 2
NS = 16
NW = NC * NS
EPW = E // NW
CH = 80
NCHUNK = EPW // CH
NPAD = 10240
RPT = NPAD // NS
ZR = 128
ZD = NPAD // NS

def _worker_id():
    return lax.axis_index("s") * NC + lax.axis_index("c")


SEG = 25
NSEG = NCHUNK // SEG


def _sc_deg_body(ei5_hbm, out_hbm, acc, seg0, seg1, ones, zbuf, lsem0,
                 lsem1, ssem):
    c = lax.axis_index("c")
    s = lax.axis_index("s")
    wid = _worker_id()
    ones_offs = list(range(0, CH - 15, 16))
    if CH % 16:
        ones_offs.append(CH - 16)
    for off in ones_offs:
        ones[pl.ds(off, 16)] = jnp.ones((16,), jnp.float32)
    for k in range(ZD // 16):
        zbuf[pl.ds(16 * k, 16)] = jnp.zeros((16,), jnp.float32)
    pltpu.sync_copy(zbuf, acc.at[pl.ds(s * ZD, ZD)])
    segs = (seg0, seg1)
    lsems = (lsem0, lsem1)
    pltpu.async_copy(ei5_hbm.at[1, wid, 0], seg0, lsem0)
    plsc.subcore_barrier()

    for b in range(NSEG):
        buf, sem = segs[b % 2], lsems[b % 2]
        if b + 1 < NSEG:
            pltpu.async_copy(
                ei5_hbm.at[1, wid, b + 1], segs[(b + 1) % 2], lsems[(b + 1) % 2]
            )
        pltpu.make_async_copy(ei5_hbm.at[1, wid, b], buf, sem).wait()

        def fire(j, carry):
            pltpu.async_copy(ones, acc.at[buf.at[j]], ssem, add=True)
            return carry

        def drain(j, carry):
            pltpu.make_async_copy(ones, acc.at[buf.at[j]], ssem).wait()
            return carry

        lax.fori_loop(0, SEG, fire, 0)
        lax.fori_loop(0, SEG, drain, 0)
    plsc.subcore_barrier()
    pltpu.sync_copy(acc.at[pl.ds(s * ZD, ZD)], out_hbm.at[c, pl.ds(s * ZD, ZD)])


def _sc_pass_body(tab_hbm, ei4_hbm, dst1_hbm, out_hbm, acc, sidx, didx0,
                  didx1, rows0, rows1, gsem0, gsem1, ssem0, ssem1, lsem0,
                  lsem1):
    c = lax.axis_index("c")
    s = lax.axis_index("s")
    wid = _worker_id()

    def zfill(i, carry):
        for k in range(D // 16):
            rows0[i, pl.ds(16 * k, 16)] = jnp.zeros((16,), jnp.float32)
        return carry

    def dst_slice(j):
        return dst1_hbm.at[pl.ds(pl.multiple_of(wid * EPW + j * CH, 8), CH)]

    lax.fori_loop(0, CH, zfill, 0)
    for r in range(RPT // CH):
        pltpu.async_copy(rows0, acc.at[pl.ds(s * RPT + r * CH, CH)], ssem0)
    pltpu.async_copy(ei4_hbm.at[0, wid], sidx, gsem1)
    pltpu.async_copy(dst_slice(0), didx0, lsem0)
    pltpu.async_copy(dst_slice(1), didx1, lsem1)
    for r in range(RPT // CH):
        pltpu.make_async_copy(
            rows0, acc.at[pl.ds(s * RPT + r * CH, CH)], ssem0
        ).wait()
    pltpu.make_async_copy(ei4_hbm.at[0, wid], sidx, gsem1).wait()
    pltpu.make_async_copy(dst_slice(0), didx0, lsem0).wait()
    pltpu.make_async_copy(dst_slice(1), didx1, lsem1).wait()
    pltpu.async_copy(tab_hbm.at[sidx.at[0]], rows0, gsem0)
    plsc.subcore_barrier()

    def gather(j, buf, sem):
        pltpu.async_copy(tab_hbm.at[sidx.at[j]], buf, sem)

    def gather_wait(j, buf, sem):
        pltpu.make_async_copy(tab_hbm.at[sidx.at[j]], buf, sem).wait()

    def scat(buf, dbuf, sem):
        pltpu.async_copy(buf, acc.at[dbuf], sem, add=True)

    def scat_wait(buf, dbuf, sem):
        pltpu.make_async_copy(buf, acc.at[dbuf], sem).wait()

    def dload(j, dbuf, sem):
        pltpu.async_copy(dst_slice(j), dbuf, sem)

    def dload_wait(j, dbuf, sem):
        pltpu.make_async_copy(dst_slice(j), dbuf, sem).wait()


    def pair(i, carry):
        j0 = 2 * i

        @pl.when(i >= 1)
        def _():
            scat_wait(rows1, didx1, ssem1)
            dload(j0 + 1, didx1, lsem1)

        gather(j0 + 1, rows1, gsem1)
        gather_wait(j0, rows0, gsem0)

        @pl.when(i >= 1)
        def _():
            dload_wait(j0, didx0, lsem0)

        scat(rows0, didx0, ssem0)
        scat_wait(rows0, didx0, ssem0)
        dload(j0 + 2, didx0, lsem0)
        gather(j0 + 2, rows0, gsem0)

        @pl.when(i >= 1)
        def _():
            dload_wait(j0 + 1, didx1, lsem1)

        gather_wait(j0 + 1, rows1, gsem1)
        scat(rows1, didx1, ssem1)
        return carry

    last = NCHUNK - 1
    lax.fori_loop(0, (NCHUNK - 1) // 2, pair, 0)
    scat_wait(rows1, didx1, ssem1)
    gather_wait(last, rows0, gsem0)
    dload_wait(last, didx0, lsem0)
    scat(rows0, didx0, ssem0)
    scat_wait(rows0, didx0, ssem0)
    plsc.subcore_barrier()
    pltpu.sync_copy(
        acc.at[pl.ds(s * RPT, RPT)], out_hbm.at[c, pl.ds(s * RPT, RPT)]
    )


@functools.lru_cache(maxsize=None)
def _get_sc_kernels():
    mesh = plsc.VectorSubcoreMesh(
        core_axis_name="c", subcore_axis_name="s",
        num_cores=NC, num_subcores=NS,
    )
    sc_deg = pl.kernel(
        _sc_deg_body,
        out_type=jax.ShapeDtypeStruct((NC, NPAD), jnp.float32),
        mesh=mesh,
        scratch_types=[
            pltpu.VMEM_SHARED((NPAD,), jnp.float32),
            pltpu.VMEM((SEG, CH), jnp.int32),
            pltpu.VMEM((SEG, CH), jnp.int32),
            pltpu.VMEM((CH,), jnp.float32),
            pltpu.VMEM((ZD,), jnp.float32),
            pltpu.SemaphoreType.DMA,
            pltpu.SemaphoreType.DMA,
            pltpu.SemaphoreType.DMA,
        ],
    )
    sc_pass = pl.kernel(
        _sc_pass_body,
        out_type=jax.ShapeDtypeStruct((NC, NPAD, D), jnp.float32),
        mesh=mesh,
        scratch_types=[
            pltpu.VMEM_SHARED((NPAD, D), jnp.float32),
            pltpu.VMEM((NCHUNK, CH), jnp.int32),
            pltpu.VMEM((CH,), jnp.int32),
            pltpu.VMEM((CH,), jnp.int32),
            pltpu.VMEM((CH, D), jnp.float32),
            pltpu.VMEM((CH, D), jnp.float32),
            pltpu.SemaphoreType.DMA,
            pltpu.SemaphoreType.DMA,
            pltpu.SemaphoreType.DMA,
            pltpu.SemaphoreType.DMA,
            pltpu.SemaphoreType.DMA,
            pltpu.SemaphoreType.DMA,
        ],
    )
    return sc_deg, sc_pass


RB = 2048
GRID = NPAD // RB
_CT = (((1,), (1,)), ((), ()))
_F32 = jnp.float32


def _tc_l1_body(x_ref, wli_ref, bli_ref, wlos_ref, blos_ref, l1_ref):
    x = x_ref[...]
    t = lax.dot_general(x, wli_ref[...], _CT, preferred_element_type=_F32)
    t = jnp.maximum(t + bli_ref[...][None, :], 0.0)
    wlos = wlos_ref[...]
    l1_ref[...] = (
        lax.dot_general(x, wlos[:, :D], _CT, preferred_element_type=_F32)
        + lax.dot_general(t, wlos[:, D:], _CT, preferred_element_type=_F32)
        + blos_ref[...][None, :]
    )


_tc_l1 = pl.pallas_call(
    _tc_l1_body,
    grid=(GRID,),
    in_specs=[
        pl.BlockSpec((RB, D), lambda b: (b, 0)),
        pl.BlockSpec((2 * D, D), lambda b: (0, 0)),
        pl.BlockSpec((2 * D,), lambda b: (0,)),
        pl.BlockSpec((D, 3 * D), lambda b: (0, 0)),
        pl.BlockSpec((D,), lambda b: (0,)),
    ],
    out_specs=pl.BlockSpec((RB, D), lambda b: (b, 0)),
    out_shape=jax.ShapeDtypeStruct((N, D), _F32),
)


def _tc_a_body(x_ref, dp_ref, wg1_ref, wg2_ref, zt_ref, dinv_ref):
    x = x_ref[...]
    deg = dp_ref[0] + dp_ref[1] + 1.0
    dinv = lax.rsqrt(deg)
    dinv_ref[...] = dinv
    wc = lax.dot_general(
        wg2_ref[...], wg1_ref[...], (((1,), (0,)), ((), ())),
        preferred_element_type=_F32,
    )
    zt_ref[...] = dinv[:, None] * lax.dot_general(
        x, wc, _CT, preferred_element_type=_F32
    )


_tc_a = pl.pallas_call(
    _tc_a_body,
    grid=(GRID,),
    in_specs=[
        pl.BlockSpec((RB, D), lambda b: (b, 0)),
        pl.BlockSpec((NC, RB), lambda b: (0, b)),
        pl.BlockSpec((2 * D, D), lambda b: (0, 0)),
        pl.BlockSpec((D, 2 * D), lambda b: (0, 0)),
    ],
    out_specs=[
        pl.BlockSpec((RB, D), lambda b: (b, 0)),
        pl.BlockSpec((RB,), lambda b: (b,)),
    ],
    out_shape=[
        jax.ShapeDtypeStruct((N, D), _F32),
        jax.ShapeDtypeStruct((NPAD,), _F32),
    ],
)


def _tc_b_body(acc_ref, zt_ref, dinv_ref, wg2_ref, bg1_ref, zt2_ref):
    dinv = dinv_ref[...][:, None]
    cvec = lax.dot_general(
        bg1_ref[...][None, :], wg2_ref[...], _CT, preferred_element_type=_F32
    )
    y1p = dinv * (acc_ref[0] + acc_ref[1] + zt_ref[...]) + cvec
    zt2_ref[...] = dinv * y1p


_tc_b = pl.pallas_call(
    _tc_b_body,
    grid=(GRID,),
    in_specs=[
        pl.BlockSpec((NC, RB, D), lambda b: (0, b, 0)),
        pl.BlockSpec((RB, D), lambda b: (b, 0)),
        pl.BlockSpec((RB,), lambda b: (b,)),
        pl.BlockSpec((D, 2 * D), lambda b: (0, 0)),
        pl.BlockSpec((2 * D,), lambda b: (0,)),
    ],
    out_specs=pl.BlockSpec((RB, D), lambda b: (b, 0)),
    out_shape=jax.ShapeDtypeStruct((N, D), _F32),
)


def _tc_c_body(acc_ref, zt2_ref, dinv_ref, xn_ref, wlo_ref, blo_ref,
               bg2_ref, x2_ref):
    dinv = dinv_ref[...][:, None]
    g2 = dinv * (acc_ref[0] + acc_ref[1] + zt2_ref[...]) + bg2_ref[...][None, :]
    wlo = wlo_ref[...]
    x2_ref[...] = (
        lax.dot_general(xn_ref[...], wlo[:, :D], _CT, preferred_element_type=_F32)
        + lax.dot_general(g2, wlo[:, D:], _CT, preferred_element_type=_F32)
        + blo_ref[...][None, :]
    )


_tc_c = pl.pallas_call(
    _tc_c_body,
    grid=(GRID,),
    in_specs=[
        pl.BlockSpec((NC, RB, D), lambda b: (0, b, 0)),
        pl.BlockSpec((RB, D), lambda b: (b, 0)),
        pl.BlockSpec((RB,), lambda b: (b,)),
        pl.BlockSpec((RB, D), lambda b: (b, 0)),
        pl.BlockSpec((D, 2 * D), lambda b: (0, 0)),
        pl.BlockSpec((D,), lambda b: (0,)),
        pl.BlockSpec((D,), lambda b: (0,)),
    ],
    out_specs=pl.BlockSpec((RB, D), lambda b: (b, 0)),
    out_shape=jax.ShapeDtypeStruct((N, D), _F32),
)


def kernel(x_self, x_neighbor, edge_index, W_lin_in, b_lin_in, W_los, b_los,
           W_g1, b_g1, W_g2, b_g2, W_lo, b_lo):
    _sc_deg, _sc_pass = _get_sc_kernels()
    ei4 = edge_index.reshape(2, NW, NCHUNK, CH)
    ei5 = edge_index.reshape(2, NW, NSEG, SEG, CH)
    dst1 = ei4[1].reshape(E)
    deg_part = _sc_deg(ei5)
    l1 = _tc_l1(x_self, W_lin_in, b_lin_in, W_los, b_los)
    zt, dinv = _tc_a(x_self, deg_part, W_g1, W_g2)
    acc1 = _sc_pass(zt, ei4, dst1)
    zt2 = _tc_b(acc1, zt, dinv, W_g2, b_g1)
    acc2 = _sc_pass(zt2, ei4, dst1)
    x2 = _tc_c(acc2, zt2, dinv, x_neighbor, W_lo, b_lo, b_g2)
    return (l1, x2)
